# baseline (speedup 1.0000x reference)
import jax
import jax.numpy as jnp
from jax.experimental import pallas as pl
from jax.experimental.pallas import tpu as pltpu

_BLK = 2048


def _tower(x, gate_W, gate_b, W_all, exp_b, cls_W, cls_b, n_exp, hid):
    xh = x.astype(jnp.bfloat16)
    logits = jnp.dot(xh, gate_W.astype(jnp.bfloat16),
                     preferred_element_type=jnp.float32) + gate_b
    logits = logits - jnp.max(logits, axis=-1, keepdims=True)
    expl = jnp.exp(logits)
    gates = expl / jnp.sum(expl, axis=-1, keepdims=True)
    eo = jnp.dot(xh, W_all.astype(jnp.bfloat16),
                 preferred_element_type=jnp.float32)
    vec = jnp.dot(gates, exp_b, preferred_element_type=jnp.float32)
    for e in range(n_exp):
        vec = vec + gates[:, e:e + 1] * eo[:, e * hid:(e + 1) * hid]
    cls = jnp.dot(vec, cls_W, preferred_element_type=jnp.float32) + cls_b
    return cls, vec


def _fused_body(n_exp, hid,
                img_ref, txt_ref,
                igW_ref, igb_ref, iWa_ref, ieb_ref, icW_ref, icb_ref,
                tgW_ref, tgb_ref, tWa_ref, teb_ref, tcW_ref, tcb_ref,
                icls_ref, tcls_ref, ivec_ref, tvec_ref):
    icls, ivec = _tower(img_ref[...], igW_ref[...], igb_ref[...], iWa_ref[...],
                        ieb_ref[...], icW_ref[...], icb_ref[...], n_exp, hid)
    icls_ref[...] = icls
    ivec_ref[...] = ivec
    tcls, tvec = _tower(txt_ref[...], tgW_ref[...], tgb_ref[...], tWa_ref[...],
                        teb_ref[...], tcW_ref[...], tcb_ref[...], n_exp, hid)
    tcls_ref[...] = tcls
    tvec_ref[...] = tvec


def kernel(image, text,
           img_gate_W, img_gate_b, img_exp_W, img_exp_b, img_cls_W, img_cls_b,
           txt_gate_W, txt_gate_b, txt_exp_W, txt_exp_b, txt_cls_W, txt_cls_b):
    b, d_img = image.shape
    _, d_txt = text.shape
    n_exp = img_gate_W.shape[1]
    hid = img_exp_W.shape[2]
    cls = img_cls_W.shape[1]

    iWa = jnp.transpose(img_exp_W, (1, 0, 2)).reshape(d_img, n_exp * hid)
    tWa = jnp.transpose(txt_exp_W, (1, 0, 2)).reshape(d_txt, n_exp * hid)
    igb = img_gate_b.reshape(1, n_exp)
    tgb = txt_gate_b.reshape(1, n_exp)
    icb = img_cls_b.reshape(1, cls)
    tcb = txt_cls_b.reshape(1, cls)

    grid = (b // _BLK,)

    def row_spec(width):
        return pl.BlockSpec((_BLK, width), lambda i: (i, 0))

    def full_spec(shape):
        return pl.BlockSpec(shape, lambda i: (0,) * len(shape))

    import functools
    body = functools.partial(_fused_body, n_exp, hid)

    out = pl.pallas_call(
        body,
        grid=grid,
        in_specs=[
            row_spec(d_img),
            row_spec(d_txt),
            full_spec((d_img, n_exp)),
            full_spec((1, n_exp)),
            full_spec((d_img, n_exp * hid)),
            full_spec((n_exp, hid)),
            full_spec((hid, cls)),
            full_spec((1, cls)),
            full_spec((d_txt, n_exp)),
            full_spec((1, n_exp)),
            full_spec((d_txt, n_exp * hid)),
            full_spec((n_exp, hid)),
            full_spec((hid, cls)),
            full_spec((1, cls)),
        ],
        out_specs=[
            row_spec(cls),
            row_spec(cls),
            row_spec(hid),
            row_spec(hid),
        ],
        out_shape=[
            jax.ShapeDtypeStruct((b, cls), jnp.float32),
            jax.ShapeDtypeStruct((b, cls), jnp.float32),
            jax.ShapeDtypeStruct((b, hid), jnp.float32),
            jax.ShapeDtypeStruct((b, hid), jnp.float32),
        ],
        compiler_params=pltpu.CompilerParams(
            dimension_semantics=("parallel",),
        ),
    )(image, text,
      img_gate_W, igb, iWa, img_exp_b, img_cls_W, icb,
      txt_gate_W, tgb, tWa, txt_exp_b, txt_cls_W, tcb)

    return (out[0], out[1], out[2], out[3])


# speedup vs baseline: 1.0359x; 1.0359x over previous
import jax
import jax.numpy as jnp
from jax.experimental import pallas as pl
from jax.experimental.pallas import tpu as pltpu

_BLK = 2048


def _tower(x, gate_W, gate_b, W_all, exp_b, cls_W, cls_b, n_exp, hid):
    logits = jnp.dot(x, gate_W, preferred_element_type=jnp.float32) + gate_b
    logits = logits - jnp.max(logits, axis=-1, keepdims=True)
    expl = jnp.exp(logits)
    gates = expl / jnp.sum(expl, axis=-1, keepdims=True)
    eo = jnp.dot(x, W_all, preferred_element_type=jnp.float32)
    vec = jnp.dot(gates, exp_b, preferred_element_type=jnp.float32)
    for e in range(n_exp):
        vec = vec + gates[:, e:e + 1] * eo[:, e * hid:(e + 1) * hid]
    cls = jnp.dot(vec, cls_W, preferred_element_type=jnp.float32) + cls_b
    return cls, vec


def _fused_body(n_exp, hid,
                img_ref, txt_ref,
                igW_ref, igb_ref, iWa_ref, ieb_ref, icW_ref, icb_ref,
                tgW_ref, tgb_ref, tWa_ref, teb_ref, tcW_ref, tcb_ref,
                icls_ref, tcls_ref, ivec_ref, tvec_ref):
    icls, ivec = _tower(img_ref[...], igW_ref[...], igb_ref[...], iWa_ref[...],
                        ieb_ref[...], icW_ref[...], icb_ref[...], n_exp, hid)
    icls_ref[...] = icls
    ivec_ref[...] = ivec
    tcls, tvec = _tower(txt_ref[...], tgW_ref[...], tgb_ref[...], tWa_ref[...],
                        teb_ref[...], tcW_ref[...], tcb_ref[...], n_exp, hid)
    tcls_ref[...] = tcls
    tvec_ref[...] = tvec


def kernel(image, text,
           img_gate_W, img_gate_b, img_exp_W, img_exp_b, img_cls_W, img_cls_b,
           txt_gate_W, txt_gate_b, txt_exp_W, txt_exp_b, txt_cls_W, txt_cls_b):
    b, d_img = image.shape
    _, d_txt = text.shape
    n_exp = img_gate_W.shape[1]
    hid = img_exp_W.shape[2]
    cls = img_cls_W.shape[1]

    iWa = jnp.transpose(img_exp_W, (1, 0, 2)).reshape(d_img, n_exp * hid)
    tWa = jnp.transpose(txt_exp_W, (1, 0, 2)).reshape(d_txt, n_exp * hid)
    igb = img_gate_b.reshape(1, n_exp)
    tgb = txt_gate_b.reshape(1, n_exp)
    icb = img_cls_b.reshape(1, cls)
    tcb = txt_cls_b.reshape(1, cls)

    grid = (b // _BLK,)

    def row_spec(width):
        return pl.BlockSpec((_BLK, width), lambda i: (i, 0))

    def full_spec(shape):
        return pl.BlockSpec(shape, lambda i: (0,) * len(shape))

    import functools
    body = functools.partial(_fused_body, n_exp, hid)

    out = pl.pallas_call(
        body,
        grid=grid,
        in_specs=[
            row_spec(d_img),
            row_spec(d_txt),
            full_spec((d_img, n_exp)),
            full_spec((1, n_exp)),
            full_spec((d_img, n_exp * hid)),
            full_spec((n_exp, hid)),
            full_spec((hid, cls)),
            full_spec((1, cls)),
            full_spec((d_txt, n_exp)),
            full_spec((1, n_exp)),
            full_spec((d_txt, n_exp * hid)),
            full_spec((n_exp, hid)),
            full_spec((hid, cls)),
            full_spec((1, cls)),
        ],
        out_specs=[
            row_spec(cls),
            row_spec(cls),
            row_spec(hid),
            row_spec(hid),
        ],
        out_shape=[
            jax.ShapeDtypeStruct((b, cls), jnp.float32),
            jax.ShapeDtypeStruct((b, cls), jnp.float32),
            jax.ShapeDtypeStruct((b, hid), jnp.float32),
            jax.ShapeDtypeStruct((b, hid), jnp.float32),
        ],
        compiler_params=pltpu.CompilerParams(
            dimension_semantics=("parallel",),
        ),
    )(image, text,
      img_gate_W, igb, iWa, img_exp_b, img_cls_W, icb,
      txt_gate_W, tgb, tWa, txt_exp_b, txt_cls_W, tcb)

    return (out[0], out[1], out[2], out[3])


# speedup vs baseline: 1.1404x; 1.1009x over previous
import jax
import jax.numpy as jnp
from jax.experimental import pallas as pl
from jax.experimental.pallas import tpu as pltpu

_BLK = 1024


def _tower(x, gate_W, gate_b, W_all, exp_b, cls_W, cls_b, n_exp, hid):
    logits = jnp.dot(x, gate_W, preferred_element_type=jnp.float32) + gate_b
    logits = logits - jnp.max(logits, axis=-1, keepdims=True)
    expl = jnp.exp(logits)
    gates = expl / jnp.sum(expl, axis=-1, keepdims=True)
    eo = jnp.dot(x, W_all, preferred_element_type=jnp.float32)
    vec = jnp.dot(gates, exp_b, preferred_element_type=jnp.float32)
    for e in range(n_exp):
        vec = vec + gates[:, e:e + 1] * eo[:, e * hid:(e + 1) * hid]
    cls = jnp.dot(vec, cls_W, preferred_element_type=jnp.float32) + cls_b
    return cls, vec


def _fused_body(n_exp, hid,
                img_ref, txt_ref,
                igW_ref, igb_ref, iWa_ref, ieb_ref, icW_ref, icb_ref,
                tgW_ref, tgb_ref, tWa_ref, teb_ref, tcW_ref, tcb_ref,
                icls_ref, tcls_ref, ivec_ref, tvec_ref):
    icls, ivec = _tower(img_ref[...], igW_ref[...], igb_ref[...], iWa_ref[...],
                        ieb_ref[...], icW_ref[...], icb_ref[...], n_exp, hid)
    icls_ref[...] = icls
    ivec_ref[...] = ivec
    tcls, tvec = _tower(txt_ref[...], tgW_ref[...], tgb_ref[...], tWa_ref[...],
                        teb_ref[...], tcW_ref[...], tcb_ref[...], n_exp, hid)
    tcls_ref[...] = tcls
    tvec_ref[...] = tvec


def kernel(image, text,
           img_gate_W, img_gate_b, img_exp_W, img_exp_b, img_cls_W, img_cls_b,
           txt_gate_W, txt_gate_b, txt_exp_W, txt_exp_b, txt_cls_W, txt_cls_b):
    b, d_img = image.shape
    _, d_txt = text.shape
    n_exp = img_gate_W.shape[1]
    hid = img_exp_W.shape[2]
    cls = img_cls_W.shape[1]

    iWa = jnp.transpose(img_exp_W, (1, 0, 2)).reshape(d_img, n_exp * hid)
    tWa = jnp.transpose(txt_exp_W, (1, 0, 2)).reshape(d_txt, n_exp * hid)
    igb = img_gate_b.reshape(1, n_exp)
    tgb = txt_gate_b.reshape(1, n_exp)
    icb = img_cls_b.reshape(1, cls)
    tcb = txt_cls_b.reshape(1, cls)

    grid = (b // _BLK,)

    def row_spec(width):
        return pl.BlockSpec((_BLK, width), lambda i: (i, 0))

    def full_spec(shape):
        return pl.BlockSpec(shape, lambda i: (0,) * len(shape))

    import functools
    body = functools.partial(_fused_body, n_exp, hid)

    out = pl.pallas_call(
        body,
        grid=grid,
        in_specs=[
            row_spec(d_img),
            row_spec(d_txt),
            full_spec((d_img, n_exp)),
            full_spec((1, n_exp)),
            full_spec((d_img, n_exp * hid)),
            full_spec((n_exp, hid)),
            full_spec((hid, cls)),
            full_spec((1, cls)),
            full_spec((d_txt, n_exp)),
            full_spec((1, n_exp)),
            full_spec((d_txt, n_exp * hid)),
            full_spec((n_exp, hid)),
            full_spec((hid, cls)),
            full_spec((1, cls)),
        ],
        out_specs=[
            row_spec(cls),
            row_spec(cls),
            row_spec(hid),
            row_spec(hid),
        ],
        out_shape=[
            jax.ShapeDtypeStruct((b, cls), jnp.float32),
            jax.ShapeDtypeStruct((b, cls), jnp.float32),
            jax.ShapeDtypeStruct((b, hid), jnp.float32),
            jax.ShapeDtypeStruct((b, hid), jnp.float32),
        ],
        compiler_params=pltpu.CompilerParams(
            dimension_semantics=("parallel",),
        ),
    )(image, text,
      img_gate_W, igb, iWa, img_exp_b, img_cls_W, icb,
      txt_gate_W, tgb, tWa, txt_exp_b, txt_cls_W, tcb)

    return (out[0], out[1], out[2], out[3])


# speedup vs baseline: 1.1597x; 1.0170x over previous
import jax
import jax.numpy as jnp
from jax.experimental import pallas as pl
from jax.experimental.pallas import tpu as pltpu

_BLK = 512


def _tower(x, gate_W, gate_b, W_all, exp_b, cls_W, cls_b, n_exp, hid):
    logits = jnp.dot(x, gate_W, preferred_element_type=jnp.float32) + gate_b
    logits = logits - jnp.max(logits, axis=-1, keepdims=True)
    expl = jnp.exp(logits)
    gates = expl / jnp.sum(expl, axis=-1, keepdims=True)
    eo = jnp.dot(x, W_all, preferred_element_type=jnp.float32)
    vec = jnp.dot(gates, exp_b, preferred_element_type=jnp.float32)
    for e in range(n_exp):
        vec = vec + gates[:, e:e + 1] * eo[:, e * hid:(e + 1) * hid]
    cls = jnp.dot(vec, cls_W, preferred_element_type=jnp.float32) + cls_b
    return cls, vec


def _fused_body(n_exp, hid,
                img_ref, txt_ref,
                igW_ref, igb_ref, iWa_ref, ieb_ref, icW_ref, icb_ref,
                tgW_ref, tgb_ref, tWa_ref, teb_ref, tcW_ref, tcb_ref,
                icls_ref, tcls_ref, ivec_ref, tvec_ref):
    icls, ivec = _tower(img_ref[...], igW_ref[...], igb_ref[...], iWa_ref[...],
                        ieb_ref[...], icW_ref[...], icb_ref[...], n_exp, hid)
    icls_ref[...] = icls
    ivec_ref[...] = ivec
    tcls, tvec = _tower(txt_ref[...], tgW_ref[...], tgb_ref[...], tWa_ref[...],
                        teb_ref[...], tcW_ref[...], tcb_ref[...], n_exp, hid)
    tcls_ref[...] = tcls
    tvec_ref[...] = tvec


def kernel(image, text,
           img_gate_W, img_gate_b, img_exp_W, img_exp_b, img_cls_W, img_cls_b,
           txt_gate_W, txt_gate_b, txt_exp_W, txt_exp_b, txt_cls_W, txt_cls_b):
    b, d_img = image.shape
    _, d_txt = text.shape
    n_exp = img_gate_W.shape[1]
    hid = img_exp_W.shape[2]
    cls = img_cls_W.shape[1]

    iWa = jnp.transpose(img_exp_W, (1, 0, 2)).reshape(d_img, n_exp * hid)
    tWa = jnp.transpose(txt_exp_W, (1, 0, 2)).reshape(d_txt, n_exp * hid)
    igb = img_gate_b.reshape(1, n_exp)
    tgb = txt_gate_b.reshape(1, n_exp)
    icb = img_cls_b.reshape(1, cls)
    tcb = txt_cls_b.reshape(1, cls)

    grid = (b // _BLK,)

    def row_spec(width):
        return pl.BlockSpec((_BLK, width), lambda i: (i, 0))

    def full_spec(shape):
        return pl.BlockSpec(shape, lambda i: (0,) * len(shape))

    import functools
    body = functools.partial(_fused_body, n_exp, hid)

    out = pl.pallas_call(
        body,
        grid=grid,
        in_specs=[
            row_spec(d_img),
            row_spec(d_txt),
            full_spec((d_img, n_exp)),
            full_spec((1, n_exp)),
            full_spec((d_img, n_exp * hid)),
            full_spec((n_exp, hid)),
            full_spec((hid, cls)),
            full_spec((1, cls)),
            full_spec((d_txt, n_exp)),
            full_spec((1, n_exp)),
            full_spec((d_txt, n_exp * hid)),
            full_spec((n_exp, hid)),
            full_spec((hid, cls)),
            full_spec((1, cls)),
        ],
        out_specs=[
            row_spec(cls),
            row_spec(cls),
            row_spec(hid),
            row_spec(hid),
        ],
        out_shape=[
            jax.ShapeDtypeStruct((b, cls), jnp.float32),
            jax.ShapeDtypeStruct((b, cls), jnp.float32),
            jax.ShapeDtypeStruct((b, hid), jnp.float32),
            jax.ShapeDtypeStruct((b, hid), jnp.float32),
        ],
        compiler_params=pltpu.CompilerParams(
            dimension_semantics=("parallel",),
        ),
    )(image, text,
      img_gate_W, igb, iWa, img_exp_b, img_cls_W, icb,
      txt_gate_W, tgb, tWa, txt_exp_b, txt_cls_W, tcb)

    return (out[0], out[1], out[2], out[3])
